# baseline (speedup 1.0000x reference)
import functools

import jax
import jax.numpy as jnp
from jax import lax
from jax.experimental import pallas as pl
from jax.experimental.pallas import tpu as pltpu
from jax.experimental.pallas import tpu_sc as plsc

D_MODEL = 64
NUM_CORES = 2
NUM_SUBCORES = 16
NUM_WORKERS = NUM_CORES * NUM_SUBCORES
BATCH = 4096
SEQ = 200
SG_IDX = 256
SG_PER_T = BATCH // SG_IDX
N_SG = SEQ * SG_PER_T
SG_PER_W = N_SG // NUM_WORKERS


@functools.cache
def _build_gather(vocab: int):
    mesh = plsc.VectorSubcoreMesh(core_axis_name="c", subcore_axis_name="s")
    idx_per_w = SG_PER_W * SG_IDX

    @functools.partial(
        pl.kernel,
        mesh=mesh,
        out_type=jax.ShapeDtypeStruct((SEQ, 8, BATCH // 128, 8, 128),
                                      jnp.float32),
        scratch_types=[
            pltpu.VMEM((idx_per_w,), jnp.int32),
            pltpu.VMEM((SG_IDX, D_MODEL), jnp.float32),
            pltpu.VMEM((SG_IDX, D_MODEL), jnp.float32),
            pltpu.VMEM((SG_IDX, D_MODEL), jnp.float32),
            pltpu.VMEM((8, 2, 9, 129), jnp.float32),
            pltpu.VMEM((8, 2, 9, 129), jnp.float32),
            pltpu.SemaphoreType.DMA,
            pltpu.SemaphoreType.DMA,
            pltpu.SemaphoreType.DMA,
            pltpu.SemaphoreType.DMA,
            pltpu.SemaphoreType.DMA,
        ],
        compiler_params=pltpu.CompilerParams(use_tc_tiling_on_sc=False,
                                             needs_layout_passes=False,
                                             disable_bounds_checks=True),
    )
    def gather_kernel(idx_hbm, table_hbm, out_hbm, idx_v, rows0, rows1,
                      rows2, obuf0, obuf1, gsem0, gsem1, gsem2,
                      osem0, osem1):
        wid = lax.axis_index("s") * NUM_CORES + lax.axis_index("c")
        sg0 = wid * SG_PER_W
        rows = (rows0, rows1, rows2)
        obuf = (obuf0, obuf1)
        gsem = (gsem0, gsem1, gsem2)
        osem = (osem0, osem1)
        iota16 = lax.iota(jnp.int32, 16)

        pltpu.sync_copy(idx_hbm.at[pl.ds(sg0 * SG_IDX, idx_per_w)], idx_v)

        def start_gather(i, b):
            pltpu.async_copy(
                table_hbm.at[idx_v.at[pl.ds(i * SG_IDX, SG_IDX)]], rows[b],
                gsem[b])

        def wait_gather(b):
            pltpu.make_async_copy(
                table_hbm.at[idx_v.at[pl.ds(0, SG_IDX)]], rows[b],
                gsem[b]).wait()

        def out_slice(i):
            s = sg0 + i
            t = s // SG_PER_T
            b0 = (s % SG_PER_T) * 2
            return out_hbm.at[t, :, pl.ds(b0, 2)]

        def obuf_view(b):
            return obuf[b].at[:, :, pl.ds(0, 8), pl.ds(0, 128)]

        def start_out(i, b):
            pltpu.async_copy(obuf_view(b), out_slice(i), osem[b])

        def wait_out(b):
            pltpu.make_async_copy(obuf_view(b), out_slice(0), osem[b]).wait()

        ddvecs = [(16 * m + iota16) >> 3 for m in range(8)]
        subvecs = [(16 * m + iota16) & 7 for m in range(8)]

        def format_group(r, o):
            rv = rows[r]
            ov = obuf[o]

            def jj_body(jj_static):
                jvec = jnp.full((16,), jj_static, jnp.int32)

                @plsc.parallel_loop(0, 128)
                def _(l):
                    lvec = jnp.full((16,), l, jnp.int32)
                    row = jj_static * 128 + l
                    vs = [rv[row, pl.ds(16 * m, 16)] for m in range(4)]
                    for m in range(4):
                        plsc.store_scatter(
                            ov, [ddvecs[m], jvec, subvecs[m], lvec], vs[m])

            jj_body(0)
            jj_body(1)

        def step(i, r, o, do_wait_out, do_gather):
            wait_gather(r)
            if do_wait_out:
                wait_out(o)
            format_group(r, o)
            start_out(i, o)
            if do_gather:
                start_gather(i + 3, r)

        for r in range(3):
            start_gather(r, r)

        for k in range(6):
            step(k, k % 3, k % 2, k >= 2, True)

        def hexad_body(p, _):
            for k in range(6):
                step(p * 6 + k, k % 3, k % 2, True, True)
            return 0

        lax.fori_loop(1, SG_PER_W // 6, hexad_body, 0)

        base = (SG_PER_W // 6) * 6
        for k in range(SG_PER_W - base):
            i = base + k
            step(i, i % 3, i % 2, True, i + 3 < SG_PER_W)
        for o in range(2):
            wait_out(o)

    return gather_kernel


def kernel(x, table):
    xt = jnp.transpose(x).reshape(-1).astype(jnp.int32)
    out5 = _build_gather(table.shape[0])(xt, table)
    o = jnp.transpose(out5, (2, 4, 0, 1, 3))
    return o.reshape(BATCH, SEQ, D_MODEL)


# speedup vs baseline: 1.0062x; 1.0062x over previous
import functools

import jax
import jax.numpy as jnp
from jax import lax
from jax.experimental import pallas as pl
from jax.experimental.pallas import tpu as pltpu
from jax.experimental.pallas import tpu_sc as plsc

D_MODEL = 64
NUM_CORES = 2
NUM_SUBCORES = 16
NUM_WORKERS = NUM_CORES * NUM_SUBCORES
BATCH = 4096
SEQ = 200
SG_IDX = 256
SG_PER_T = BATCH // SG_IDX
N_SG = SEQ * SG_PER_T
SG_PER_W = N_SG // NUM_WORKERS


@functools.cache
def _build_gather(vocab: int):
    mesh = plsc.VectorSubcoreMesh(core_axis_name="c", subcore_axis_name="s")
    idx_per_w = SG_PER_W * SG_IDX

    @functools.partial(
        pl.kernel,
        mesh=mesh,
        out_type=jax.ShapeDtypeStruct((SEQ, 8, BATCH // 128, 8, 128),
                                      jnp.float32),
        scratch_types=[
            pltpu.VMEM((idx_per_w,), jnp.int32),
            pltpu.VMEM((SG_IDX, D_MODEL), jnp.float32),
            pltpu.VMEM((SG_IDX, D_MODEL), jnp.float32),
            pltpu.VMEM((8, 2, 9, 129), jnp.float32),
            pltpu.VMEM((8, 2, 9, 129), jnp.float32),
            pltpu.SemaphoreType.DMA,
            pltpu.SemaphoreType.DMA,
            pltpu.SemaphoreType.DMA,
            pltpu.SemaphoreType.DMA,
        ],
        compiler_params=pltpu.CompilerParams(use_tc_tiling_on_sc=False,
                                             needs_layout_passes=False,
                                             disable_bounds_checks=True),
    )
    def gather_kernel(idx_hbm, table_hbm, out_hbm, idx_v, rows0, rows1,
                      obuf0, obuf1, gsem0, gsem1, osem0, osem1):
        wid = lax.axis_index("s") * NUM_CORES + lax.axis_index("c")
        sg0 = wid * SG_PER_W
        rows = (rows0, rows1)
        obuf = (obuf0, obuf1)
        gsem = (gsem0, gsem1)
        osem = (osem0, osem1)
        iota16 = lax.iota(jnp.int32, 16)

        pltpu.sync_copy(idx_hbm.at[pl.ds(sg0 * SG_IDX, idx_per_w)], idx_v)

        def start_gather(i, b):
            pltpu.async_copy(
                table_hbm.at[idx_v.at[pl.ds(i * SG_IDX, SG_IDX)]], rows[b],
                gsem[b])

        def wait_gather(b):
            pltpu.make_async_copy(
                table_hbm.at[idx_v.at[pl.ds(0, SG_IDX)]], rows[b],
                gsem[b]).wait()

        def out_slice(i):
            s = sg0 + i
            t = s // SG_PER_T
            b0 = (s % SG_PER_T) * 2
            return out_hbm.at[t, :, pl.ds(b0, 2)]

        def obuf_view(b):
            return obuf[b].at[:, :, pl.ds(0, 8), pl.ds(0, 128)]

        def start_out(i, b):
            pltpu.async_copy(obuf_view(b), out_slice(i), osem[b])

        def wait_out(b):
            pltpu.make_async_copy(obuf_view(b), out_slice(0), osem[b]).wait()

        ddvecs = [(16 * m + iota16) >> 3 for m in range(8)]
        subvecs = [(16 * m + iota16) & 7 for m in range(8)]

        def format_group(b):
            rv = rows[b]
            ov = obuf[b]

            def jj_body(jj_static):
                jvec = jnp.full((16,), jj_static, jnp.int32)

                @plsc.parallel_loop(0, 128)
                def _(l):
                    lvec = jnp.full((16,), l, jnp.int32)
                    row = jj_static * 128 + l
                    vs = [rv[row, pl.ds(16 * m, 16)] for m in range(4)]
                    for m in range(4):
                        plsc.store_scatter(
                            ov, [ddvecs[m], jvec, subvecs[m], lvec], vs[m])

            jj_body(0)
            jj_body(1)

        start_gather(0, 0)
        start_gather(1, 1)

        for b in range(2):
            wait_gather(b)
            format_group(b)
            start_out(b, b)
            start_gather(b + 2, b)

        def pair_body(p, _):
            for b in range(2):
                i = p * 2 + b
                wait_gather(b)
                wait_out(b)
                format_group(b)
                start_out(i, b)
                start_gather(i + 2, b)
            return 0

        lax.fori_loop(1, SG_PER_W // 2 - 1, pair_body, 0)

        for b in range(2):
            i = SG_PER_W - 2 + b
            wait_gather(b)
            wait_out(b)
            format_group(b)
            start_out(i, b)
        for b in range(2):
            wait_out(b)

    return gather_kernel


def kernel(x, table):
    xt = jnp.transpose(x).reshape(-1).astype(jnp.int32)
    out5 = _build_gather(table.shape[0])(xt, table)
    o = jnp.transpose(out5, (2, 4, 0, 1, 3))
    return o.reshape(BATCH, SEQ, D_MODEL)
